# Initial kernel scaffold; baseline (speedup 1.0000x reference)
import functools

import jax
import jax.numpy as jnp
from jax import lax
from jax.experimental import pallas as pl
from jax.experimental.pallas import tpu as pltpu
from jax.experimental.pallas import tpu_sc as plsc

N_USER = 5000
N_ITEM = 5000
N = N_USER + N_ITEM
E = 320000
D = 128
N_LAYERS = 3

NC = 2
NS = 16
NW = NC * NS
CH = 128
CPW = -(-E // (NW * CH))
E_PAD = NW * CPW * CH
RPS = 626
N_PAD = RPS * NS
EP_CH = RPS // 2


def _layer_body(col_ref, row_ref, val_ref, cur_ref, part_ref,
                col_v, row_v, val_v, gbuf, obuf, acc_sh, sem):
    c = lax.axis_index("c")
    s = lax.axis_index("s")
    wid = s * NC + c

    zeros16 = jnp.zeros((16,), jnp.float32)

    def _zero(i, carry):
        for g in range(8):
            obuf[i, pl.ds(g * 16, 16)] = zeros16
        return carry
    lax.fori_loop(0, EP_CH, _zero, 0)
    r0 = s * RPS
    pltpu.sync_copy(obuf, acc_sh.at[pl.ds(r0, EP_CH)])
    pltpu.sync_copy(obuf, acc_sh.at[pl.ds(r0 + EP_CH, EP_CH)])
    plsc.subcore_barrier()

    base = wid * CPW
    pltpu.sync_copy(col_ref.at[pl.ds(base, CPW)], col_v)
    pltpu.sync_copy(row_ref.at[pl.ds(base, CPW)], row_v)
    pltpu.sync_copy(val_ref.at[pl.ds(base, CPW)], val_v)

    e16 = lax.iota(jnp.int32, 16)
    eidx = [e16 + (g * 16) for g in range(8)]

    def _chunk(j, carry):
        pltpu.async_copy(cur_ref.at[col_v.at[j]], gbuf, sem).wait()
        vals = [val_v[j, pl.ds(g * 16, 16)] for g in range(8)]

        def _scale(d, inner):
            d16 = jnp.full((16,), d, jnp.int32)
            for g in range(8):
                x = plsc.load_gather(gbuf, [eidx[g], d16])
                plsc.store_scatter(gbuf, [eidx[g], d16], x * vals[g])
            return inner
        lax.fori_loop(0, D, _scale, 0)
        pltpu.sync_copy(gbuf, acc_sh.at[row_v.at[j]], add=True)
        return carry
    lax.fori_loop(0, CPW, _chunk, 0)

    plsc.subcore_barrier()

    for k in range(2):
        rk = r0 + k * EP_CH
        pltpu.sync_copy(acc_sh.at[pl.ds(rk, EP_CH)], obuf)
        pltpu.sync_copy(obuf, part_ref.at[c, pl.ds(rk, EP_CH)])


def _spmm_layer(col2d, row2d, val2d, cur):
    mesh = plsc.VectorSubcoreMesh(core_axis_name="c", subcore_axis_name="s",
                                  num_cores=NC, num_subcores=NS)
    return pl.kernel(
        _layer_body,
        out_type=jax.ShapeDtypeStruct((NC, N_PAD, D), jnp.float32),
        mesh=mesh,
        scratch_types=[
            pltpu.VMEM((CPW, CH), jnp.int32),
            pltpu.VMEM((CPW, CH), jnp.int32),
            pltpu.VMEM((CPW, CH), jnp.float32),
            pltpu.VMEM((CH, D), jnp.float32),
            pltpu.VMEM((EP_CH, D), jnp.float32),
            pltpu.VMEM_SHARED((N_PAD, D), jnp.float32),
            pltpu.SemaphoreType.DMA,
        ],
    )(col2d, row2d, val2d, cur)


def _combine_body(p0_ref, p1_ref, acc_ref, cur_out, acc_out):
    s = p0_ref[...] + p1_ref[...]
    cur_out[...] = s
    acc_out[...] = acc_ref[...] + s


def _combine(p0, p1, acc):
    blk = 16
    grid = (N_PAD // blk,)
    bs = pl.BlockSpec((blk, D), lambda i: (i, 0))
    return pl.pallas_call(
        _combine_body,
        grid=grid,
        in_specs=[bs, bs, bs],
        out_specs=[bs, bs],
        out_shape=[jax.ShapeDtypeStruct((N_PAD, D), jnp.float32),
                   jax.ShapeDtypeStruct((N_PAD, D), jnp.float32)],
    )(p0, p1, acc)


def kernel(edge_index, adj_values, uEmbeds, iEmbeds):
    row = edge_index[0].astype(jnp.int32)
    col = edge_index[1].astype(jnp.int32)
    val = adj_values.astype(jnp.float32)
    pad = E_PAD - E
    col2d = jnp.pad(col, (0, pad)).reshape(NW * CPW, CH)
    row2d = jnp.pad(row, (0, pad)).reshape(NW * CPW, CH)
    val2d = jnp.pad(val, (0, pad)).reshape(NW * CPW, CH)

    embeds = jnp.concatenate([uEmbeds, iEmbeds], axis=0)
    embeds = jnp.pad(embeds, ((0, N_PAD - N), (0, 0)))
    acc = embeds
    cur = embeds
    for _ in range(N_LAYERS):
        parts = _spmm_layer(col2d, row2d, val2d, cur)
        cur, acc = _combine(parts[0], parts[1], acc)
    return acc[:N_USER], acc[N_USER:N]


# speedup vs baseline: 1.7007x; 1.7007x over previous
import functools

import jax
import jax.numpy as jnp
from jax import lax
from jax.experimental import pallas as pl
from jax.experimental.pallas import tpu as pltpu
from jax.experimental.pallas import tpu_sc as plsc

N_USER = 5000
N_ITEM = 5000
N = N_USER + N_ITEM
E = 320000
D = 128
N_LAYERS = 3

NC = 2
NS = 16
CH = 128
CPS = 160
CPB = 40
NB = CPS // CPB
E_PAD = NS * CPS * CH
HALF = 5120
N_PAD = NC * HALF
DUMMY = HALF
RPS = HALF // NS


def _layer_body(col_ref, row_ref, val_ref, cur_ref, out_ref,
                col_v, row_v, val_v, gbuf, obuf, acc_sh, sem):
    c = lax.axis_index("c")
    s = lax.axis_index("s")
    lo = c * HALF

    zeros16 = jnp.zeros((16,), jnp.float32)

    def _zero(i, carry):
        for g in range(8):
            obuf[i, pl.ds(g * 16, 16)] = zeros16
        return carry
    lax.fori_loop(0, RPS // 2, _zero, 0)
    pltpu.sync_copy(obuf, acc_sh.at[pl.ds(s * RPS, RPS // 2)])
    pltpu.sync_copy(obuf, acc_sh.at[pl.ds(s * RPS + RPS // 2, RPS // 2)])

    @pl.when(s == 0)
    def _zero_dummy():
        pltpu.sync_copy(obuf.at[pl.ds(0, 8)], acc_sh.at[pl.ds(DUMMY, 8)])

    plsc.subcore_barrier()

    def _block(b, bcarry):
        base = s * CPS + b * CPB
        pltpu.sync_copy(col_ref.at[pl.ds(base, CPB)], col_v)
        pltpu.sync_copy(row_ref.at[pl.ds(base, CPB)], row_v)
        pltpu.sync_copy(val_ref.at[pl.ds(base, CPB)], val_v)

        def _chunk(j, carry):
            pltpu.async_copy(cur_ref.at[col_v.at[j]], gbuf, sem).wait()

            def _scale(g, inner):
                sl = pl.ds(g * 16, 16)
                row16 = row_v[j, sl]
                owned = (row16 >= lo) & (row16 < lo + HALF)
                vals16 = jnp.where(owned, val_v[j, sl], 0.0)
                row_v[j, sl] = jnp.where(owned, row16 - lo, DUMMY)
                e0 = g * 16
                for e in range(16):
                    v = vals16[e]
                    for k in range(8):
                        ssl = pl.ds(k * 16, 16)
                        gbuf[e0 + e, ssl] = gbuf[e0 + e, ssl] * v
                return inner
            lax.fori_loop(0, CH // 16, _scale, 0)

            pltpu.sync_copy(gbuf, acc_sh.at[row_v.at[j]], add=True)
            return carry
        lax.fori_loop(0, CPB, _chunk, 0)
        return bcarry
    lax.fori_loop(0, NB, _block, 0)

    plsc.subcore_barrier()

    for h in range(2):
        r0 = s * RPS + h * (RPS // 2)
        pltpu.sync_copy(acc_sh.at[pl.ds(r0, RPS // 2)], obuf)
        pltpu.sync_copy(obuf, out_ref.at[pl.ds(lo + r0, RPS // 2)])


def _spmm_layer(col2d, row2d, val2d, cur):
    mesh = plsc.VectorSubcoreMesh(core_axis_name="c", subcore_axis_name="s",
                                  num_cores=NC, num_subcores=NS)
    return pl.kernel(
        _layer_body,
        out_type=jax.ShapeDtypeStruct((N_PAD, D), jnp.float32),
        mesh=mesh,
        scratch_types=[
            pltpu.VMEM((CPB, CH), jnp.int32),
            pltpu.VMEM((CPB, CH), jnp.int32),
            pltpu.VMEM((CPB, CH), jnp.float32),
            pltpu.VMEM((CH, D), jnp.float32),
            pltpu.VMEM((RPS // 2, D), jnp.float32),
            pltpu.VMEM_SHARED((HALF + 8, D), jnp.float32),
            pltpu.SemaphoreType.DMA,
        ],
    )(col2d, row2d, val2d, cur)


def _acc_body(p_ref, acc_ref, acc_out):
    acc_out[...] = acc_ref[...] + p_ref[...]


def _accumulate(p, acc):
    blk = 256
    bs = pl.BlockSpec((blk, D), lambda i: (i, 0))
    return pl.pallas_call(
        _acc_body,
        grid=(N_PAD // blk,),
        in_specs=[bs, bs],
        out_specs=bs,
        out_shape=jax.ShapeDtypeStruct((N_PAD, D), jnp.float32),
    )(p, acc)


def kernel(edge_index, adj_values, uEmbeds, iEmbeds):
    row = edge_index[0].astype(jnp.int32)
    col = edge_index[1].astype(jnp.int32)
    val = adj_values.astype(jnp.float32)
    pad = E_PAD - E
    col2d = jnp.pad(col, (0, pad)).reshape(NS * CPS, CH)
    row2d = jnp.pad(row, (0, pad)).reshape(NS * CPS, CH)
    val2d = jnp.pad(val, (0, pad)).reshape(NS * CPS, CH)

    embeds = jnp.concatenate([uEmbeds, iEmbeds], axis=0)
    embeds = jnp.pad(embeds, ((0, N_PAD - N), (0, 0)))
    acc = embeds
    cur = embeds
    for _ in range(N_LAYERS):
        cur = _spmm_layer(col2d, row2d, val2d, cur)
        acc = _accumulate(cur, acc)
    return acc[:N_USER], acc[N_USER:N]


# speedup vs baseline: 1.8668x; 1.0976x over previous
import functools

import jax
import jax.numpy as jnp
from jax import lax
from jax.experimental import pallas as pl
from jax.experimental.pallas import tpu as pltpu
from jax.experimental.pallas import tpu_sc as plsc

N_USER = 5000
N_ITEM = 5000
N = N_USER + N_ITEM
E = 320000
D = 128
N_LAYERS = 3

NC = 2
NS = 16
CH = 128
CPS = 160
CPB = 40
NB = CPS // CPB
E_PAD = NS * CPS * CH
HALF = 5120
N_PAD = NC * HALF
DUMMY = HALF
RPS = HALF // NS


def _layer_body(col_ref, row_ref, val_ref, cur_ref, out_ref,
                col_v, row_v, val_v, gbuf, obuf, acc_sh, sem0, sem1):
    c = lax.axis_index("c")
    s = lax.axis_index("s")
    lo = c * HALF

    zeros16 = jnp.zeros((16,), jnp.float32)

    def _zero(i, carry):
        for g in range(8):
            obuf[i, pl.ds(g * 16, 16)] = zeros16
        return carry
    lax.fori_loop(0, RPS // 2, _zero, 0)
    pltpu.sync_copy(obuf, acc_sh.at[pl.ds(s * RPS, RPS // 2)])
    pltpu.sync_copy(obuf, acc_sh.at[pl.ds(s * RPS + RPS // 2, RPS // 2)])

    @pl.when(s == 0)
    def _zero_dummy():
        pltpu.sync_copy(obuf.at[pl.ds(0, 8)], acc_sh.at[pl.ds(DUMMY, 8)])

    plsc.subcore_barrier()

    sems = (sem0, sem1)

    def _fire(j, b):
        pltpu.async_copy(cur_ref.at[col_v.at[j]], gbuf.at[b], sems[b])

    def _drain(j, b):
        pltpu.make_async_copy(cur_ref.at[col_v.at[j]], gbuf.at[b], sems[b]).wait()

    def _process(j, b):
        def _scale(g, inner):
            sl = pl.ds(g * 16, 16)
            row16 = row_v[j, sl]
            owned = (row16 >= lo) & (row16 < lo + HALF)
            vals16 = jnp.where(owned, val_v[j, sl], 0.0)
            row_v[j, sl] = jnp.where(owned, row16 - lo, DUMMY)
            e0 = g * 16
            for e in range(16):
                v = vals16[e]
                for k in range(8):
                    ssl = pl.ds(k * 16, 16)
                    gbuf[b, e0 + e, ssl] = gbuf[b, e0 + e, ssl] * v
            return inner
        lax.fori_loop(0, CH // 16, _scale, 0)
        pltpu.sync_copy(gbuf.at[b], acc_sh.at[row_v.at[j]], add=True)

    def _block(b, bcarry):
        base = s * CPS + b * CPB
        pltpu.sync_copy(col_ref.at[pl.ds(base, CPB)], col_v)
        pltpu.sync_copy(row_ref.at[pl.ds(base, CPB)], row_v)
        pltpu.sync_copy(val_ref.at[pl.ds(base, CPB)], val_v)

        _fire(0, 0)

        def _pair(j2, carry):
            for p in range(2):
                j = j2 * 2 + p

                @pl.when(j + 1 < CPB)
                def _next():
                    _fire(j + 1, (p + 1) % 2)
                _drain(j, p)
                _process(j, p)
            return carry
        lax.fori_loop(0, CPB // 2, _pair, 0)
        return bcarry
    lax.fori_loop(0, NB, _block, 0)

    plsc.subcore_barrier()

    for h in range(2):
        r0 = s * RPS + h * (RPS // 2)
        pltpu.sync_copy(acc_sh.at[pl.ds(r0, RPS // 2)], obuf)
        pltpu.sync_copy(obuf, out_ref.at[pl.ds(lo + r0, RPS // 2)])


def _spmm_layer(col2d, row2d, val2d, cur):
    mesh = plsc.VectorSubcoreMesh(core_axis_name="c", subcore_axis_name="s",
                                  num_cores=NC, num_subcores=NS)
    return pl.kernel(
        _layer_body,
        out_type=jax.ShapeDtypeStruct((N_PAD, D), jnp.float32),
        mesh=mesh,
        scratch_types=[
            pltpu.VMEM((CPB, CH), jnp.int32),
            pltpu.VMEM((CPB, CH), jnp.int32),
            pltpu.VMEM((CPB, CH), jnp.float32),
            pltpu.VMEM((2, CH, D), jnp.float32),
            pltpu.VMEM((RPS // 2, D), jnp.float32),
            pltpu.VMEM_SHARED((HALF + 8, D), jnp.float32),
            pltpu.SemaphoreType.DMA,
            pltpu.SemaphoreType.DMA,
        ],
    )(col2d, row2d, val2d, cur)


def _acc_body(p_ref, acc_ref, acc_out):
    acc_out[...] = acc_ref[...] + p_ref[...]


def _accumulate(p, acc):
    blk = 256
    bs = pl.BlockSpec((blk, D), lambda i: (i, 0))
    return pl.pallas_call(
        _acc_body,
        grid=(N_PAD // blk,),
        in_specs=[bs, bs],
        out_specs=bs,
        out_shape=jax.ShapeDtypeStruct((N_PAD, D), jnp.float32),
    )(p, acc)


def kernel(edge_index, adj_values, uEmbeds, iEmbeds):
    row = edge_index[0].astype(jnp.int32)
    col = edge_index[1].astype(jnp.int32)
    val = adj_values.astype(jnp.float32)
    pad = E_PAD - E
    col2d = jnp.pad(col, (0, pad)).reshape(NS * CPS, CH)
    row2d = jnp.pad(row, (0, pad)).reshape(NS * CPS, CH)
    val2d = jnp.pad(val, (0, pad)).reshape(NS * CPS, CH)

    embeds = jnp.concatenate([uEmbeds, iEmbeds], axis=0)
    embeds = jnp.pad(embeds, ((0, N_PAD - N), (0, 0)))
    acc = embeds
    cur = embeds
    for _ in range(N_LAYERS):
        cur = _spmm_layer(col2d, row2d, val2d, cur)
        acc = _accumulate(cur, acc)
    return acc[:N_USER], acc[N_USER:N]
